# Initial kernel scaffold; baseline (speedup 1.0000x reference)
import jax
import jax.numpy as jnp
from jax.experimental import pallas as pl


def kernel(x, embeddings):
    raise NotImplementedError("write your pallas kernel here")


# speedup vs baseline: 1.6839x; 1.6839x over previous
import functools

import jax
import jax.numpy as jnp
from jax import lax
from jax.experimental import pallas as pl
from jax.experimental.pallas import tpu as pltpu
from jax.experimental.pallas import tpu_sc as plsc

_INFO = plsc.get_sparse_core_info()
_NC = _INFO.num_cores
_NS = _INFO.num_subcores
_NW = _NC * _NS

_B = 16384 * 50
_D = 64
_CHUNK = 128
_B_PER_W = _B // _NW
_ROWS_PER_W = _B_PER_W // _CHUNK

_mesh = plsc.VectorSubcoreMesh(core_axis_name="c", subcore_axis_name="s")


@functools.partial(
    pl.kernel,
    out_type=jax.ShapeDtypeStruct((_B, _D), jnp.float32),
    mesh=_mesh,
    compiler_params=pltpu.CompilerParams(use_tc_tiling_on_sc=False),
    scratch_types=[
        pltpu.VMEM((_ROWS_PER_W, _CHUNK), jnp.int32),
        pltpu.VMEM((_CHUNK, _D), jnp.float32),
        pltpu.SemaphoreType.DMA,
    ],
)
def _embed_kernel(idx_hbm, table_hbm, out_hbm, idx_v, rows_v, sem):
    wid = lax.axis_index("s") * _NC + lax.axis_index("c")
    row_base = wid * _ROWS_PER_W
    out_base = wid * _B_PER_W

    pltpu.sync_copy(idx_hbm.at[pl.ds(row_base, _ROWS_PER_W)], idx_v)

    def body(g, _):
        pltpu.async_copy(table_hbm.at[idx_v.at[g]], rows_v, sem).wait()
        pltpu.sync_copy(rows_v, out_hbm.at[pl.ds(out_base + g * _CHUNK, _CHUNK)])
        return ()

    lax.fori_loop(0, _ROWS_PER_W, body, ())


def kernel(x, embeddings):
    idx2d = x.reshape(_B // _CHUNK, _CHUNK).astype(jnp.int32)
    out = _embed_kernel(idx2d, embeddings)
    return out.reshape(x.shape[0], x.shape[1], _D)


# speedup vs baseline: 1.8768x; 1.1145x over previous
import functools

import jax
import jax.numpy as jnp
from jax import lax
from jax.experimental import pallas as pl
from jax.experimental.pallas import tpu as pltpu
from jax.experimental.pallas import tpu_sc as plsc

_INFO = plsc.get_sparse_core_info()
_NC = _INFO.num_cores
_NS = _INFO.num_subcores
_NW = _NC * _NS

_B = 16384 * 50
_D = 64
_IW = 128
_RPC = 2
_CROWS = _RPC * _IW
_B_PER_W = _B // _NW
_IDX_ROWS = _B_PER_W // _IW
_NCH = _B_PER_W // _CROWS
_NBUF = 3
_LOOK = 2

_mesh = plsc.VectorSubcoreMesh(core_axis_name="c", subcore_axis_name="s")


@functools.partial(
    pl.kernel,
    out_type=jax.ShapeDtypeStruct((_B, _D), jnp.float32),
    mesh=_mesh,
    compiler_params=pltpu.CompilerParams(use_tc_tiling_on_sc=False),
    scratch_types=[
        pltpu.VMEM((_IDX_ROWS, _IW), jnp.int32),
        pltpu.VMEM((_NBUF, _CROWS, _D), jnp.float32),
        pltpu.SemaphoreType.DMA,
        pltpu.SemaphoreType.DMA,
        pltpu.SemaphoreType.DMA,
        pltpu.SemaphoreType.DMA,
        pltpu.SemaphoreType.DMA,
        pltpu.SemaphoreType.DMA,
    ],
)
def _embed_kernel(idx_hbm, table_hbm, out_hbm, idx_v, rows_v,
                  g0, g1, g2, s0, s1, s2):
    gsem = (g0, g1, g2)
    ssem = (s0, s1, s2)
    wid = lax.axis_index("s") * _NC + lax.axis_index("c")
    row_base = wid * _IDX_ROWS
    out_base = wid * _B_PER_W

    pltpu.sync_copy(idx_hbm.at[pl.ds(row_base, _IDX_ROWS)], idx_v)

    def fire_gather(ch, b):
        for r in range(_RPC):
            pltpu.async_copy(
                table_hbm.at[idx_v.at[ch * _RPC + r]],
                rows_v.at[b].at[pl.ds(r * _IW, _IW)],
                gsem[b],
            )

    def drain_gather(ch, b):
        for r in range(_RPC):
            pltpu.make_async_copy(
                table_hbm.at[idx_v.at[ch * _RPC + r]],
                rows_v.at[b].at[pl.ds(r * _IW, _IW)],
                gsem[b],
            ).wait()

    def fire_store(ch, b):
        pltpu.async_copy(
            rows_v.at[b], out_hbm.at[pl.ds(out_base + ch * _CROWS, _CROWS)],
            ssem[b],
        )

    def drain_store(ch, b):
        pltpu.make_async_copy(
            rows_v.at[b], out_hbm.at[pl.ds(out_base + ch * _CROWS, _CROWS)],
            ssem[b],
        ).wait()

    for ch in range(_LOOK):
        fire_gather(ch, ch % _NBUF)

    def body(i, _):
        for bb in range(_NBUF):
            g = i * _NBUF + bb
            drain_gather(g, bb)
            fire_store(g, bb)
            nb = (bb + _LOOK) % _NBUF

            @pl.when(jnp.logical_and(g >= 1, g + _LOOK < _NCH))
            def _():
                drain_store(g - 1, nb)

            @pl.when(g + _LOOK < _NCH)
            def _():
                fire_gather(g + _LOOK, nb)
        return ()

    lax.fori_loop(0, _NCH // _NBUF, body, ())

    g = _NCH - 1
    bb = g % _NBUF
    drain_gather(g, bb)
    fire_store(g, bb)

    for ch in range(_NCH - 3, _NCH):
        drain_store(ch, ch % _NBUF)


def kernel(x, embeddings):
    idx2d = x.reshape(_B // _IW, _IW).astype(jnp.int32)
    out = _embed_kernel(idx2d, embeddings)
    return out.reshape(x.shape[0], x.shape[1], _D)
